# Initial kernel scaffold; baseline (speedup 1.0000x reference)
import jax
import jax.numpy as jnp
from jax.experimental import pallas as pl


def kernel(input_len, pe_table):
    raise NotImplementedError("write your pallas kernel here")


# speedup vs baseline: 1.7354x; 1.7354x over previous
import functools

import jax
import jax.numpy as jnp
from jax import lax
from jax.experimental import pallas as pl
from jax.experimental.pallas import tpu as pltpu
from jax.experimental.pallas import tpu_sc as plsc

D_MODEL = 1024
MAX_SEQ_LEN = 4096
BATCH = 8
ROWS = BATCH * MAX_SEQ_LEN
C = 32
NCHUNKS = ROWS // C
NW = 32
CHUNKS_PER_WORKER = NCHUNKS // NW

_mesh = plsc.VectorSubcoreMesh(core_axis_name="c", subcore_axis_name="s")


@functools.partial(
    pl.kernel,
    out_type=jax.ShapeDtypeStruct((ROWS, D_MODEL), jnp.float32),
    mesh=_mesh,
    scratch_types=[
        pltpu.VMEM((32,), jnp.int32),
        pltpu.VMEM((C,), jnp.int32),
        pltpu.VMEM((C, D_MODEL), jnp.float32),
        pltpu.VMEM((C, D_MODEL), jnp.float32),
        pltpu.SemaphoreType.DMA,
    ],
)
def _pe_lookup(len_hbm, pe_hbm, out_hbm, len_v, idx_v, dbuf, zbuf, sem):
    wid = lax.axis_index("s") * 2 + lax.axis_index("c")

    pltpu.sync_copy(len_hbm, len_v.at[pl.ds(0, BATCH)])

    zeros = jnp.zeros((16,), jnp.int32)
    for h in range(C // 16):
        idx_v[pl.ds(h * 16, 16)] = zeros
    pltpu.async_copy(pe_hbm.at[idx_v], zbuf, sem).wait()

    def body(i, carry):
        ci = wid + i * NW
        r0 = pl.multiple_of(ci * C, C)
        b = r0 >> 12
        p0 = pl.multiple_of(r0 & (MAX_SEQ_LEN - 1), C)
        l = len_v[pl.ds(b, 16)][0]
        nd = jnp.clip(l - p0, 0, C)

        @pl.when(nd == 0)
        def _():
            pltpu.sync_copy(zbuf, out_hbm.at[pl.ds(r0, C)])

        @pl.when(nd > 0)
        def _():
            lane = lax.iota(jnp.int32, 16)
            for h in range(C // 16):
                j = lane + h * 16
                idx_v[pl.ds(h * 16, 16)] = jnp.where(j < nd, p0 + 1 + j, 0)
            pltpu.async_copy(pe_hbm.at[idx_v], dbuf, sem).wait()
            pltpu.sync_copy(dbuf, out_hbm.at[pl.ds(r0, C)])

        return carry

    lax.fori_loop(0, CHUNKS_PER_WORKER, body, 0)


def kernel(input_len, pe_table):
    out = _pe_lookup(input_len, pe_table)
    return out.reshape(BATCH, MAX_SEQ_LEN, D_MODEL)


# speedup vs baseline: 1.8458x; 1.0636x over previous
import functools

import jax
import jax.numpy as jnp
from jax import lax
from jax.experimental import pallas as pl
from jax.experimental.pallas import tpu as pltpu
from jax.experimental.pallas import tpu_sc as plsc

D_MODEL = 1024
MAX_SEQ_LEN = 4096
BATCH = 8
ROWS = BATCH * MAX_SEQ_LEN
C = 32
NCHUNKS = ROWS // C
NW = 32
CHUNKS_PER_WORKER = NCHUNKS // NW

_mesh = plsc.VectorSubcoreMesh(core_axis_name="c", subcore_axis_name="s")


@functools.partial(
    pl.kernel,
    out_type=jax.ShapeDtypeStruct((ROWS, D_MODEL), jnp.float32),
    mesh=_mesh,
    scratch_types=[
        pltpu.VMEM((32,), jnp.int32),
        pltpu.VMEM((C,), jnp.int32),
        pltpu.VMEM((C, D_MODEL), jnp.float32),
        pltpu.VMEM((C, D_MODEL), jnp.float32),
        pltpu.VMEM((C, D_MODEL), jnp.float32),
        pltpu.SemaphoreType.DMA,
        pltpu.SemaphoreType.DMA,
        pltpu.SemaphoreType.DMA,
    ],
)
def _pe_lookup(len_hbm, pe_hbm, out_hbm, len_v, idx_v, dbuf0, dbuf1, zbuf,
               rsem, wsem0, wsem1):
    wid = lax.axis_index("s") * 2 + lax.axis_index("c")

    pltpu.sync_copy(len_hbm, len_v.at[pl.ds(0, BATCH)])

    zeros = jnp.zeros((16,), jnp.int32)
    for h in range(C // 16):
        idx_v[pl.ds(h * 16, 16)] = zeros
    pltpu.async_copy(pe_hbm.at[idx_v], zbuf, rsem).wait()

    def body(i, carry):
        for par, dbuf, wsem in ((0, dbuf0, wsem0), (1, dbuf1, wsem1)):
            ci = wid + (2 * i + par) * NW
            r0 = pl.multiple_of(ci * C, C)
            b = r0 >> 12
            p0 = pl.multiple_of(r0 & (MAX_SEQ_LEN - 1), C)
            l = len_v[pl.ds(b, 16)][0]
            nd = jnp.clip(l - p0, 0, C)

            @pl.when(i >= 1)
            def _():
                pltpu.make_async_copy(dbuf, out_hbm.at[pl.ds(0, C)],
                                      wsem).wait()

            @pl.when(nd == 0)
            def _():
                pltpu.async_copy(zbuf, out_hbm.at[pl.ds(r0, C)], wsem)

            @pl.when(nd > 0)
            def _():
                lane = lax.iota(jnp.int32, 16)
                for h in range(C // 16):
                    j = lane + h * 16
                    idx_v[pl.ds(h * 16, 16)] = jnp.where(j < nd, p0 + 1 + j, 0)
                pltpu.async_copy(pe_hbm.at[idx_v], dbuf, rsem).wait()
                pltpu.async_copy(dbuf, out_hbm.at[pl.ds(r0, C)], wsem)

        return carry

    lax.fori_loop(0, CHUNKS_PER_WORKER // 2, body, 0)

    pltpu.make_async_copy(dbuf0, out_hbm.at[pl.ds(0, C)], wsem0).wait()
    pltpu.make_async_copy(dbuf1, out_hbm.at[pl.ds(0, C)], wsem1).wait()


def kernel(input_len, pe_table):
    out = _pe_lookup(input_len, pe_table)
    return out.reshape(BATCH, MAX_SEQ_LEN, D_MODEL)


# speedup vs baseline: 1.9063x; 1.0327x over previous
import functools

import jax
import jax.numpy as jnp
from jax import lax
from jax.experimental import pallas as pl
from jax.experimental.pallas import tpu as pltpu
from jax.experimental.pallas import tpu_sc as plsc

D_MODEL = 1024
MAX_SEQ_LEN = 4096
BATCH = 8
ROWS = BATCH * MAX_SEQ_LEN
C = 32
NCHUNKS = ROWS // C
NW = 32
CHUNKS_PER_WORKER = NCHUNKS // NW

_mesh = plsc.VectorSubcoreMesh(core_axis_name="c", subcore_axis_name="s")


@functools.partial(
    pl.kernel,
    out_type=jax.ShapeDtypeStruct((ROWS, D_MODEL), jnp.float32),
    mesh=_mesh,
    scratch_types=[
        pltpu.VMEM((32,), jnp.int32),
        pltpu.VMEM((C,), jnp.int32),
        pltpu.VMEM((C,), jnp.int32),
        pltpu.VMEM((C, D_MODEL), jnp.float32),
        pltpu.VMEM((C, D_MODEL), jnp.float32),
        pltpu.VMEM((C, D_MODEL), jnp.float32),
        pltpu.SemaphoreType.DMA,
        pltpu.SemaphoreType.DMA,
        pltpu.SemaphoreType.DMA,
        pltpu.SemaphoreType.DMA,
    ],
)
def _pe_lookup(len_hbm, pe_hbm, out_hbm, len_v, idx0, idx1, dbuf0, dbuf1,
               zbuf, rsem0, rsem1, wsem0, wsem1):
    wid = lax.axis_index("s") * 2 + lax.axis_index("c")

    pltpu.sync_copy(len_hbm, len_v.at[pl.ds(0, BATCH)])

    zeros = jnp.zeros((16,), jnp.int32)
    for h in range(C // 16):
        idx0[pl.ds(h * 16, 16)] = zeros
    pltpu.async_copy(pe_hbm.at[idx0], zbuf, rsem0).wait()

    idx = (idx0, idx1)
    dbuf = (dbuf0, dbuf1)
    rsem = (rsem0, rsem1)
    wsem = (wsem0, wsem1)
    lane = lax.iota(jnp.int32, 16)

    def chunk_params(k):
        r0 = pl.multiple_of((wid + k * NW) * C, C)
        b = r0 >> 12
        p0 = pl.multiple_of(r0 & (MAX_SEQ_LEN - 1), C)
        l = len_v[pl.ds(b, 16)][0]
        nd = jnp.clip(l - p0, 0, C)
        return r0, p0, nd

    def launch_gather(k, par):
        r0, p0, nd = chunk_params(k)

        @pl.when(nd > 0)
        def _():
            for h in range(C // 16):
                j = lane + h * 16
                idx[par][pl.ds(h * 16, 16)] = jnp.where(j < nd, p0 + 1 + j, 0)
            pltpu.async_copy(pe_hbm.at[idx[par]], dbuf[par], rsem[par])

    def launch_store(k, par):
        r0, p0, nd = chunk_params(k)

        @pl.when(nd > 0)
        def _():
            pltpu.make_async_copy(pe_hbm.at[idx[par]], dbuf[par],
                                  rsem[par]).wait()
            pltpu.async_copy(dbuf[par], out_hbm.at[pl.ds(r0, C)], wsem[par])

        @pl.when(nd == 0)
        def _():
            pltpu.async_copy(zbuf, out_hbm.at[pl.ds(r0, C)], wsem[par])

    def body(i, carry):
        for par in (0, 1):
            k = 2 * i + par

            @pl.when(k >= 2)
            def _():
                pltpu.make_async_copy(dbuf[par], out_hbm.at[pl.ds(0, C)],
                                      wsem[par]).wait()

            launch_gather(k, par)

            if par == 0:
                @pl.when(k >= 1)
                def _():
                    launch_store(k - 1, 1)
            else:
                launch_store(k - 1, 0)
        return carry

    lax.fori_loop(0, CHUNKS_PER_WORKER // 2, body, 0)

    launch_store(CHUNKS_PER_WORKER - 1, (CHUNKS_PER_WORKER - 1) & 1)
    pltpu.make_async_copy(dbuf0, out_hbm.at[pl.ds(0, C)], wsem0).wait()
    pltpu.make_async_copy(dbuf1, out_hbm.at[pl.ds(0, C)], wsem1).wait()


def kernel(input_len, pe_table):
    out = _pe_lookup(input_len, pe_table)
    return out.reshape(BATCH, MAX_SEQ_LEN, D_MODEL)


# speedup vs baseline: 2.2824x; 1.1973x over previous
import functools

import jax
import jax.numpy as jnp
from jax import lax
from jax.experimental import pallas as pl
from jax.experimental.pallas import tpu as pltpu
from jax.experimental.pallas import tpu_sc as plsc

D_MODEL = 1024
MAX_SEQ_LEN = 4096
BATCH = 8
ROWS = BATCH * MAX_SEQ_LEN
C = 16
NCHUNKS = ROWS // C
NW = 32
CHUNKS_PER_WORKER = NCHUNKS // NW

_mesh = plsc.VectorSubcoreMesh(core_axis_name="c", subcore_axis_name="s")


@functools.partial(
    pl.kernel,
    out_type=jax.ShapeDtypeStruct((ROWS, D_MODEL), jnp.float32),
    mesh=_mesh,
    scratch_types=[
        pltpu.VMEM((32,), jnp.int32),
        pltpu.VMEM((C,), jnp.int32),
        pltpu.VMEM((C,), jnp.int32),
        pltpu.VMEM((C, D_MODEL), jnp.float32),
        pltpu.VMEM((C, D_MODEL), jnp.float32),
        pltpu.VMEM((C, D_MODEL), jnp.float32),
        pltpu.SemaphoreType.DMA,
        pltpu.SemaphoreType.DMA,
        pltpu.SemaphoreType.DMA,
        pltpu.SemaphoreType.DMA,
    ],
)
def _pe_lookup(len_hbm, pe_hbm, out_hbm, len_v, idx0, idx1, dbuf0, dbuf1,
               zbuf, rsem0, rsem1, wsem0, wsem1):
    wid = lax.axis_index("s") * 2 + lax.axis_index("c")

    pltpu.sync_copy(len_hbm, len_v.at[pl.ds(0, BATCH)])

    zeros = jnp.zeros((16,), jnp.int32)
    for h in range(C // 16):
        idx0[pl.ds(h * 16, 16)] = zeros
    pltpu.async_copy(pe_hbm.at[idx0], zbuf, rsem0).wait()

    idx = (idx0, idx1)
    dbuf = (dbuf0, dbuf1)
    rsem = (rsem0, rsem1)
    wsem = (wsem0, wsem1)
    lane = lax.iota(jnp.int32, 16)

    def chunk_params(k):
        r0 = pl.multiple_of((wid + k * NW) * C, C)
        b = r0 >> 12
        p0 = pl.multiple_of(r0 & (MAX_SEQ_LEN - 1), C)
        l = len_v[pl.ds(b, 16)][0]
        nd = jnp.clip(l - p0, 0, C)
        return r0, p0, nd

    def launch_gather(k, par):
        r0, p0, nd = chunk_params(k)

        @pl.when(nd > 0)
        def _():
            for h in range(C // 16):
                j = lane + h * 16
                idx[par][pl.ds(h * 16, 16)] = jnp.where(j < nd, p0 + 1 + j, 0)
            pltpu.async_copy(pe_hbm.at[idx[par]], dbuf[par], rsem[par])

    def launch_store(k, par):
        r0, p0, nd = chunk_params(k)

        @pl.when(nd > 0)
        def _():
            pltpu.make_async_copy(pe_hbm.at[idx[par]], dbuf[par],
                                  rsem[par]).wait()
            pltpu.async_copy(dbuf[par], out_hbm.at[pl.ds(r0, C)], wsem[par])

        @pl.when(nd == 0)
        def _():
            pltpu.async_copy(zbuf, out_hbm.at[pl.ds(r0, C)], wsem[par])

    def body(i, carry):
        for par in (0, 1):
            k = 2 * i + par

            @pl.when(k >= 2)
            def _():
                pltpu.make_async_copy(dbuf[par], out_hbm.at[pl.ds(0, C)],
                                      wsem[par]).wait()

            launch_gather(k, par)

            if par == 0:
                @pl.when(k >= 1)
                def _():
                    launch_store(k - 1, 1)
            else:
                launch_store(k - 1, 0)
        return carry

    lax.fori_loop(0, CHUNKS_PER_WORKER // 2, body, 0)

    launch_store(CHUNKS_PER_WORKER - 1, (CHUNKS_PER_WORKER - 1) & 1)
    pltpu.make_async_copy(dbuf0, out_hbm.at[pl.ds(0, C)], wsem0).wait()
    pltpu.make_async_copy(dbuf1, out_hbm.at[pl.ds(0, C)], wsem1).wait()


def kernel(input_len, pe_table):
    out = _pe_lookup(input_len, pe_table)
    return out.reshape(BATCH, MAX_SEQ_LEN, D_MODEL)
